# Initial kernel scaffold; baseline (speedup 1.0000x reference)
import jax
import jax.numpy as jnp
from jax import lax
from jax.experimental import pallas as pl
from jax.experimental.pallas import tpu as pltpu
from jax.experimental.pallas import tpu_sc as plsc

_NC = 2
_NS = 16
_EB = 80


def _segment_sum_sc(x, src, dst, with_cnt):
    n, d = x.shape
    e = src.shape[0]
    nw = _NC * _NS
    epw = e // nw
    steps = epw // _EB
    assert epw * nw == e and steps * _EB == epw and n % (8 * _NS) == 0

    rows_per_sub = n // _NS
    nchunks = n // 8

    mesh = plsc.VectorSubcoreMesh(core_axis_name="c", subcore_axis_name="s")
    out_type = [jax.ShapeDtypeStruct((_NC, n, d), jnp.float32)]
    scratch = [
        pltpu.VMEM((_EB,), jnp.int32),
        pltpu.VMEM((_EB,), jnp.int32),
        pltpu.VMEM((_EB, d), jnp.float32),
        pltpu.VMEM((8, d), jnp.float32),
        pltpu.VMEM_SHARED((n, d), jnp.float32),
    ]
    if with_cnt:
        out_type.append(jax.ShapeDtypeStruct((_NC, n, 16), jnp.float32))
        scratch += [
            pltpu.VMEM((_EB, 16), jnp.float32),
            pltpu.VMEM((8, 16), jnp.float32),
            pltpu.VMEM_SHARED((n, 16), jnp.float32),
        ]

    def body(x_hbm, src_hbm, dst_hbm, *refs):
        if with_cnt:
            (acc_out, cnt_out, src_v, dst_v, rows_v, zero_v, acc_s,
             ones_v, zero16_v, cnt_s) = refs
        else:
            (acc_out, src_v, dst_v, rows_v, zero_v, acc_s) = refs
        c = lax.axis_index("c")
        s = lax.axis_index("s")

        for i in range(8):
            for j in range(d // 16):
                zero_v.at[pl.ds(i, 1), pl.ds(j * 16, 16)][...] = (
                    jnp.zeros((1, 16), jnp.float32))
        if with_cnt:
            for i in range(8):
                zero16_v.at[pl.ds(i, 1), pl.ds(0, 16)][...] = (
                    jnp.zeros((1, 16), jnp.float32))

            @pl.loop(0, _EB)
            def _(i):
                ones_v.at[pl.ds(i, 1), pl.ds(0, 16)][...] = (
                    jnp.ones((1, 16), jnp.float32))

        @pl.loop(s, nchunks, step=_NS)
        def _(ch):
            pltpu.sync_copy(zero_v, acc_s.at[pl.ds(ch * 8, 8)])
            if with_cnt:
                pltpu.sync_copy(zero16_v, cnt_s.at[pl.ds(ch * 8, 8)])

        plsc.subcore_barrier()

        base0 = (c * _NS + s) * epw

        @pl.loop(0, steps)
        def _(t):
            base = base0 + t * _EB
            pltpu.sync_copy(src_hbm.at[pl.ds(base, _EB)], src_v)
            pltpu.sync_copy(dst_hbm.at[pl.ds(base, _EB)], dst_v)
            pltpu.sync_copy(x_hbm.at[src_v], rows_v)
            pltpu.sync_copy(rows_v, acc_s.at[dst_v], add=True)
            if with_cnt:
                pltpu.sync_copy(ones_v, cnt_s.at[dst_v], add=True)

        plsc.subcore_barrier()

        r0 = s * rows_per_sub
        pltpu.sync_copy(acc_s.at[pl.ds(r0, rows_per_sub)],
                        acc_out.at[c].at[pl.ds(r0, rows_per_sub)])
        if with_cnt:
            pltpu.sync_copy(cnt_s.at[pl.ds(r0, rows_per_sub)],
                            cnt_out.at[c].at[pl.ds(r0, rows_per_sub)])

    return pl.kernel(body, out_type=out_type, mesh=mesh,
                     scratch_types=scratch)(x, src, dst)


def _sage_dense(x, aggp, cntp, Wl, bl, Wr, br, blk):
    n, d = x.shape
    h = Wl.shape[1]

    def body(x_ref, agg_ref, cnt_ref, wl_ref, bl_ref, wr_ref, br_ref, o_ref):
        cnt = cnt_ref[0, :, 0:1] + cnt_ref[1, :, 0:1]
        z = (agg_ref[0] + agg_ref[1]) / jnp.maximum(cnt, 1.0)
        z1 = (jnp.dot(x_ref[...], wl_ref[...], preferred_element_type=jnp.float32)
              + bl_ref[...]
              + jnp.dot(z, wr_ref[...], preferred_element_type=jnp.float32)
              + br_ref[...])
        nrm = jnp.sqrt(jnp.sum(z1 * z1, axis=1, keepdims=True))
        o_ref[...] = jnp.maximum(z1 / jnp.maximum(nrm, 1e-12), 0.0)

    return pl.pallas_call(
        body,
        grid=(n // blk,),
        in_specs=[
            pl.BlockSpec((blk, d), lambda i: (i, 0)),
            pl.BlockSpec((_NC, blk, d), lambda i: (0, i, 0)),
            pl.BlockSpec((_NC, blk, 16), lambda i: (0, i, 0)),
            pl.BlockSpec((d, h), lambda i: (0, 0)),
            pl.BlockSpec((1, h), lambda i: (0, 0)),
            pl.BlockSpec((d, h), lambda i: (0, 0)),
            pl.BlockSpec((1, h), lambda i: (0, 0)),
        ],
        out_specs=pl.BlockSpec((blk, h), lambda i: (i, 0)),
        out_shape=jax.ShapeDtypeStruct((n, h), jnp.float32),
    )(x, aggp, cntp, Wl, bl.reshape(1, -1), Wr, br.reshape(1, -1))


def _sage_dense_post(x, aggp, cntp, Wl, bl, Wr, br, Wp1, bp1, Wp2, bp2, blk):
    n, d = x.shape
    h = Wl.shape[1]
    out = Wp2.shape[1]

    def body(x_ref, agg_ref, cnt_ref, wl_ref, bl_ref, wr_ref, br_ref,
             wp1_ref, bp1_ref, wp2_ref, bp2_ref, o_ref):
        cnt = cnt_ref[0, :, 0:1] + cnt_ref[1, :, 0:1]
        z = (agg_ref[0] + agg_ref[1]) / jnp.maximum(cnt, 1.0)
        z1 = (jnp.dot(x_ref[...], wl_ref[...], preferred_element_type=jnp.float32)
              + bl_ref[...]
              + jnp.dot(z, wr_ref[...], preferred_element_type=jnp.float32)
              + br_ref[...])
        nrm = jnp.sqrt(jnp.sum(z1 * z1, axis=1, keepdims=True))
        x2 = jnp.maximum(z1 / jnp.maximum(nrm, 1e-12), 0.0)
        y = (jnp.dot(x2, wp1_ref[...], preferred_element_type=jnp.float32)
             + bp1_ref[...])
        o_ref[...] = (jnp.dot(y, wp2_ref[...], preferred_element_type=jnp.float32)
                      + bp2_ref[...])

    return pl.pallas_call(
        body,
        grid=(n // blk,),
        in_specs=[
            pl.BlockSpec((blk, d), lambda i: (i, 0)),
            pl.BlockSpec((_NC, blk, d), lambda i: (0, i, 0)),
            pl.BlockSpec((_NC, blk, 16), lambda i: (0, i, 0)),
            pl.BlockSpec((d, h), lambda i: (0, 0)),
            pl.BlockSpec((1, h), lambda i: (0, 0)),
            pl.BlockSpec((d, h), lambda i: (0, 0)),
            pl.BlockSpec((1, h), lambda i: (0, 0)),
            pl.BlockSpec((h, h), lambda i: (0, 0)),
            pl.BlockSpec((1, h), lambda i: (0, 0)),
            pl.BlockSpec((h, out), lambda i: (0, 0)),
            pl.BlockSpec((1, out), lambda i: (0, 0)),
        ],
        out_specs=pl.BlockSpec((blk, out), lambda i: (i, 0)),
        out_shape=jax.ShapeDtypeStruct((n, out), jnp.float32),
    )(x, aggp, cntp, Wl, bl.reshape(1, -1), Wr, br.reshape(1, -1),
      Wp1, bp1.reshape(1, -1), Wp2, bp2.reshape(1, -1))


def kernel(data, edge_index, W_l0, b_l0, W_r0, b_r0, W_l1, b_l1, W_r1, b_r1,
           W_p1, b_p1, W_p2, b_p2):
    src = edge_index[0]
    dst = edge_index[1]

    agg0, cnt = _segment_sum_sc(data, src, dst, with_cnt=True)
    x1 = _sage_dense(data, agg0, cnt, W_l0, b_l0, W_r0, b_r0, blk=2000)
    (agg1,) = _segment_sum_sc(x1, src, dst, with_cnt=False)
    return _sage_dense_post(x1, agg1, cnt, W_l1, b_l1, W_r1, b_r1,
                            W_p1, b_p1, W_p2, b_p2, blk=2000)


# speedup vs baseline: 3.3524x; 3.3524x over previous
import jax
import jax.numpy as jnp
from jax import lax
from jax.experimental import pallas as pl
from jax.experimental.pallas import tpu as pltpu
from jax.experimental.pallas import tpu_sc as plsc

_NC = 2
_NS = 16
_EB = 80


def _segment_sum_sc(x, src, dst, with_cnt):
    n, d = x.shape
    e = src.shape[0]
    nw = _NC * _NS
    epw = e // nw
    steps = epw // _EB
    assert epw * nw == e and steps * _EB == epw and n % 8 == 0 and n % _NS == 0

    assert n % 80 == 0
    nchunks = n // 8

    mesh = plsc.VectorSubcoreMesh(core_axis_name="c", subcore_axis_name="s")
    out_type = [jax.ShapeDtypeStruct((_NC, n, d), jnp.float32)]
    scratch = [
        pltpu.VMEM((1, _EB), jnp.int32),
        pltpu.VMEM((1, _EB), jnp.int32),
        pltpu.VMEM((_EB, d), jnp.float32),
        pltpu.VMEM((8, d), jnp.float32),
        pltpu.VMEM_SHARED((n, d), jnp.float32),
    ]
    if with_cnt:
        out_type.append(jax.ShapeDtypeStruct((_NC, n, 16), jnp.float32))
        scratch += [
            pltpu.VMEM((_EB, 16), jnp.float32),
            pltpu.VMEM((8, 16), jnp.float32),
            pltpu.VMEM_SHARED((n, 16), jnp.float32),
        ]

    def body(x_hbm, src_hbm, dst_hbm, *refs):
        if with_cnt:
            (acc_out, cnt_out, src_v, dst_v, rows_v, zero_v, acc_s,
             ones_v, zero16_v, cnt_s) = refs
        else:
            (acc_out, src_v, dst_v, rows_v, zero_v, acc_s) = refs
        c = lax.axis_index("c")
        s = lax.axis_index("s")

        for i in range(8):
            for j in range(d // 16):
                zero_v.at[pl.ds(i, 1), pl.ds(j * 16, 16)][...] = (
                    jnp.zeros((1, 16), jnp.float32))
        if with_cnt:
            for i in range(8):
                zero16_v.at[pl.ds(i, 1), pl.ds(0, 16)][...] = (
                    jnp.zeros((1, 16), jnp.float32))

            @pl.loop(0, _EB)
            def _(i):
                ones_v.at[pl.ds(i, 1), pl.ds(0, 16)][...] = (
                    jnp.ones((1, 16), jnp.float32))

        @pl.loop(s, nchunks, step=_NS)
        def _(ch):
            pltpu.sync_copy(zero_v, acc_s.at[pl.ds(ch * 8, 8)])
            if with_cnt:
                pltpu.sync_copy(zero16_v, cnt_s.at[pl.ds(ch * 8, 8)])

        plsc.subcore_barrier()

        base0 = (c * _NS + s) * epw

        @pl.loop(0, steps)
        def _(t):
            base = base0 + t * _EB
            pltpu.sync_copy(src_hbm.at[pl.ds(base, _EB)], src_v.at[0])
            pltpu.sync_copy(dst_hbm.at[pl.ds(base, _EB)], dst_v.at[0])
            pltpu.sync_copy(x_hbm.at[src_v.at[0]], rows_v)
            pltpu.sync_copy(rows_v, acc_s.at[dst_v.at[0]], add=True)
            if with_cnt:
                pltpu.sync_copy(ones_v, cnt_s.at[dst_v.at[0]], add=True)

        plsc.subcore_barrier()

        @pl.loop(s, n // 80, step=_NS)
        def _(ch):
            r0 = ch * 80
            pltpu.sync_copy(acc_s.at[pl.ds(r0, 80)],
                            acc_out.at[c].at[pl.ds(r0, 80)])
            if with_cnt:
                pltpu.sync_copy(cnt_s.at[pl.ds(r0, 80)],
                                cnt_out.at[c].at[pl.ds(r0, 80)])

    res = pl.kernel(body, out_type=out_type, mesh=mesh,
                    scratch_types=scratch)(x, src, dst)
    return res if isinstance(res, (list, tuple)) else (res,)


def _sage_dense(x, aggp, cntp, Wl, bl, Wr, br, blk):
    n, d = x.shape
    h = Wl.shape[1]

    def body(x_ref, agg_ref, cnt_ref, wl_ref, bl_ref, wr_ref, br_ref, o_ref):
        cnt = cnt_ref[0, :, 0:1] + cnt_ref[1, :, 0:1]
        z = (agg_ref[0] + agg_ref[1]) / jnp.maximum(cnt, 1.0)
        z1 = (jnp.dot(x_ref[...], wl_ref[...], preferred_element_type=jnp.float32)
              + bl_ref[...]
              + jnp.dot(z, wr_ref[...], preferred_element_type=jnp.float32)
              + br_ref[...])
        nrm = jnp.sqrt(jnp.sum(z1 * z1, axis=1, keepdims=True))
        o_ref[...] = jnp.maximum(z1 / jnp.maximum(nrm, 1e-12), 0.0)

    return pl.pallas_call(
        body,
        grid=(n // blk,),
        in_specs=[
            pl.BlockSpec((blk, d), lambda i: (i, 0)),
            pl.BlockSpec((_NC, blk, d), lambda i: (0, i, 0)),
            pl.BlockSpec((_NC, blk, 16), lambda i: (0, i, 0)),
            pl.BlockSpec((d, h), lambda i: (0, 0)),
            pl.BlockSpec((1, h), lambda i: (0, 0)),
            pl.BlockSpec((d, h), lambda i: (0, 0)),
            pl.BlockSpec((1, h), lambda i: (0, 0)),
        ],
        out_specs=pl.BlockSpec((blk, h), lambda i: (i, 0)),
        out_shape=jax.ShapeDtypeStruct((n, h), jnp.float32),
    )(x, aggp, cntp, Wl, bl.reshape(1, -1), Wr, br.reshape(1, -1))


def _sage_dense_post(x, aggp, cntp, Wl, bl, Wr, br, Wp1, bp1, Wp2, bp2, blk):
    n, d = x.shape
    h = Wl.shape[1]
    out = Wp2.shape[1]

    def body(x_ref, agg_ref, cnt_ref, wl_ref, bl_ref, wr_ref, br_ref,
             wp1_ref, bp1_ref, wp2_ref, bp2_ref, o_ref):
        cnt = cnt_ref[0, :, 0:1] + cnt_ref[1, :, 0:1]
        z = (agg_ref[0] + agg_ref[1]) / jnp.maximum(cnt, 1.0)
        z1 = (jnp.dot(x_ref[...], wl_ref[...], preferred_element_type=jnp.float32)
              + bl_ref[...]
              + jnp.dot(z, wr_ref[...], preferred_element_type=jnp.float32)
              + br_ref[...])
        nrm = jnp.sqrt(jnp.sum(z1 * z1, axis=1, keepdims=True))
        x2 = jnp.maximum(z1 / jnp.maximum(nrm, 1e-12), 0.0)
        y = (jnp.dot(x2, wp1_ref[...], preferred_element_type=jnp.float32)
             + bp1_ref[...])
        o_ref[...] = (jnp.dot(y, wp2_ref[...], preferred_element_type=jnp.float32)
                      + bp2_ref[...])

    return pl.pallas_call(
        body,
        grid=(n // blk,),
        in_specs=[
            pl.BlockSpec((blk, d), lambda i: (i, 0)),
            pl.BlockSpec((_NC, blk, d), lambda i: (0, i, 0)),
            pl.BlockSpec((_NC, blk, 16), lambda i: (0, i, 0)),
            pl.BlockSpec((d, h), lambda i: (0, 0)),
            pl.BlockSpec((1, h), lambda i: (0, 0)),
            pl.BlockSpec((d, h), lambda i: (0, 0)),
            pl.BlockSpec((1, h), lambda i: (0, 0)),
            pl.BlockSpec((h, h), lambda i: (0, 0)),
            pl.BlockSpec((1, h), lambda i: (0, 0)),
            pl.BlockSpec((h, out), lambda i: (0, 0)),
            pl.BlockSpec((1, out), lambda i: (0, 0)),
        ],
        out_specs=pl.BlockSpec((blk, out), lambda i: (i, 0)),
        out_shape=jax.ShapeDtypeStruct((n, out), jnp.float32),
    )(x, aggp, cntp, Wl, bl.reshape(1, -1), Wr, br.reshape(1, -1),
      Wp1, bp1.reshape(1, -1), Wp2, bp2.reshape(1, -1))


def kernel(data, edge_index, W_l0, b_l0, W_r0, b_r0, W_l1, b_l1, W_r1, b_r1,
           W_p1, b_p1, W_p2, b_p2):
    src = edge_index[0]
    dst = edge_index[1]

    n = data.shape[0]
    cnt_j = jax.ops.segment_sum(jnp.ones((src.shape[0],), jnp.float32), dst,
                                num_segments=n)
    cnt1 = jnp.broadcast_to(cnt_j[None, :, None], (1, n, 16))
    cnt = jnp.concatenate([cnt1, jnp.zeros_like(cnt1)], axis=0)

    (agg0,) = _segment_sum_sc(data, src, dst, with_cnt=False)
    x1 = _sage_dense(data, agg0, cnt, W_l0, b_l0, W_r0, b_r0, blk=2000)
    (agg1,) = _segment_sum_sc(x1, src, dst, with_cnt=False)
    return _sage_dense_post(x1, agg1, cnt, W_l1, b_l1, W_r1, b_r1,
                            W_p1, b_p1, W_p2, b_p2, blk=2000)


# speedup vs baseline: 5.1197x; 1.5272x over previous
import dataclasses

import jax
import jax.numpy as jnp
from jax import lax
from jax.experimental import pallas as pl
from jax.experimental.pallas import tpu as pltpu
from jax.experimental.pallas import tpu_sc as plsc

_NC = 2
_NS = 16
_EB = 80
_HR = 8
_HC = 2048


def _segment_sum_sc(x, src, dst, with_cnt):
    n, d = x.shape
    e = src.shape[0]
    nw = _NC * _NS
    epw = e // nw
    steps = epw // _EB
    assert epw * nw == e and steps * _EB == epw and n % 8 == 0 and n % _NS == 0

    assert n % 80 == 0
    nchunks = n // 8

    mesh = plsc.VectorSubcoreMesh(core_axis_name="c", subcore_axis_name="s")
    out_type = [jax.ShapeDtypeStruct((_NC, n, d), jnp.float32)]
    scratch = [
        pltpu.VMEM((1, _EB), jnp.int32),
        pltpu.VMEM((1, _EB), jnp.int32),
        pltpu.VMEM((_EB, d), jnp.float32),
        pltpu.VMEM((8, d), jnp.float32),
        pltpu.VMEM_SHARED((n, d), jnp.float32),
    ]
    if with_cnt:
        assert n <= _HR * _HC
        out_type.append(
            jax.ShapeDtypeStruct((nw * _HR, _HC), jnp.float32))
        scratch.append(pltpu.VMEM((_HR, _HC), jnp.float32))

    def body(x_hbm, src_hbm, dst_hbm, *refs):
        if with_cnt:
            (acc_out, cnt_out, src_v, dst_v, rows_v, zero_v, acc_s,
             hist_v) = refs
        else:
            (acc_out, src_v, dst_v, rows_v, zero_v, acc_s) = refs
        c = lax.axis_index("c")
        s = lax.axis_index("s")

        for i in range(8):
            for j in range(d // 16):
                zero_v.at[i, pl.ds(j * 16, 16)][...] = jnp.zeros((16,), jnp.float32)
        if with_cnt:
            for i in range(_HR):
                @pl.loop(0, _HC // 16)
                def _(j, i=i):
                    hist_v.at[i, pl.ds(j * 16, 16)][...] = (
                        jnp.zeros((16,), jnp.float32))

        @pl.loop(s, nchunks, step=_NS)
        def _(ch):
            pltpu.sync_copy(zero_v, acc_s.at[pl.ds(ch * 8, 8)])

        plsc.subcore_barrier()

        base0 = (c * _NS + s) * epw

        @pl.loop(0, steps)
        def _(t):
            base = base0 + t * _EB
            pltpu.sync_copy(src_hbm.at[pl.ds(base, _EB)], src_v.at[0])
            pltpu.sync_copy(dst_hbm.at[pl.ds(base, _EB)], dst_v.at[0])
            pltpu.sync_copy(x_hbm.at[src_v.at[0]], rows_v)
            pltpu.sync_copy(rows_v, acc_s.at[dst_v.at[0]], add=True)
            if with_cnt:
                ones16 = jnp.ones((16,), jnp.float32)
                for k in range(_EB // 16):
                    idx = dst_v.at[0, pl.ds(k * 16, 16)][...]
                    plsc.addupdate_scatter(
                        hist_v,
                        [lax.shift_right_logical(idx, 11),
                         lax.bitwise_and(idx, _HC - 1)],
                        ones16)

        plsc.subcore_barrier()

        @pl.loop(s, n // 80, step=_NS)
        def _(ch):
            r0 = ch * 80
            pltpu.sync_copy(acc_s.at[pl.ds(r0, 80)],
                            acc_out.at[c].at[pl.ds(r0, 80)])
        if with_cnt:
            w = c * _NS + s
            pltpu.sync_copy(hist_v, cnt_out.at[pl.ds(w * _HR, _HR)])

    cp = pltpu.CompilerParams()
    if "needs_layout_passes" in pltpu.CompilerParams.__dataclass_fields__:
        cp = dataclasses.replace(cp, needs_layout_passes=False)
    res = pl.kernel(body, out_type=out_type, mesh=mesh,
                    scratch_types=scratch, compiler_params=cp)(x, src, dst)
    return res if isinstance(res, (list, tuple)) else (res,)


def _sage_dense(x, aggp, cntp, Wl, bl, Wr, br, blk):
    n, d = x.shape
    h = Wl.shape[1]

    nw = cntp.shape[1]

    def body(x_ref, agg_ref, cnt_ref, wl_ref, bl_ref, wr_ref, br_ref, o_ref):
        cnt = jnp.sum(cnt_ref[...], axis=1, keepdims=True)
        z = (agg_ref[0] + agg_ref[1]) / jnp.maximum(cnt, 1.0)
        z1 = (jnp.dot(x_ref[...], wl_ref[...], preferred_element_type=jnp.float32)
              + bl_ref[...]
              + jnp.dot(z, wr_ref[...], preferred_element_type=jnp.float32)
              + br_ref[...])
        nrm = jnp.sqrt(jnp.sum(z1 * z1, axis=1, keepdims=True))
        o_ref[...] = jnp.maximum(z1 / jnp.maximum(nrm, 1e-12), 0.0)

    return pl.pallas_call(
        body,
        grid=(n // blk,),
        in_specs=[
            pl.BlockSpec((blk, d), lambda i: (i, 0)),
            pl.BlockSpec((_NC, blk, d), lambda i: (0, i, 0)),
            pl.BlockSpec((blk, nw), lambda i: (i, 0)),
            pl.BlockSpec((d, h), lambda i: (0, 0)),
            pl.BlockSpec((1, h), lambda i: (0, 0)),
            pl.BlockSpec((d, h), lambda i: (0, 0)),
            pl.BlockSpec((1, h), lambda i: (0, 0)),
        ],
        out_specs=pl.BlockSpec((blk, h), lambda i: (i, 0)),
        out_shape=jax.ShapeDtypeStruct((n, h), jnp.float32),
    )(x, aggp, cntp, Wl, bl.reshape(1, -1), Wr, br.reshape(1, -1))


def _sage_dense_post(x, aggp, cntp, Wl, bl, Wr, br, Wp1, bp1, Wp2, bp2, blk):
    n, d = x.shape
    h = Wl.shape[1]
    out = Wp2.shape[1]

    nw = cntp.shape[1]

    def body(x_ref, agg_ref, cnt_ref, wl_ref, bl_ref, wr_ref, br_ref,
             wp1_ref, bp1_ref, wp2_ref, bp2_ref, o_ref):
        cnt = jnp.sum(cnt_ref[...], axis=1, keepdims=True)
        z = (agg_ref[0] + agg_ref[1]) / jnp.maximum(cnt, 1.0)
        z1 = (jnp.dot(x_ref[...], wl_ref[...], preferred_element_type=jnp.float32)
              + bl_ref[...]
              + jnp.dot(z, wr_ref[...], preferred_element_type=jnp.float32)
              + br_ref[...])
        nrm = jnp.sqrt(jnp.sum(z1 * z1, axis=1, keepdims=True))
        x2 = jnp.maximum(z1 / jnp.maximum(nrm, 1e-12), 0.0)
        y = (jnp.dot(x2, wp1_ref[...], preferred_element_type=jnp.float32)
             + bp1_ref[...])
        o_ref[...] = (jnp.dot(y, wp2_ref[...], preferred_element_type=jnp.float32)
                      + bp2_ref[...])

    return pl.pallas_call(
        body,
        grid=(n // blk,),
        in_specs=[
            pl.BlockSpec((blk, d), lambda i: (i, 0)),
            pl.BlockSpec((_NC, blk, d), lambda i: (0, i, 0)),
            pl.BlockSpec((blk, nw), lambda i: (i, 0)),
            pl.BlockSpec((d, h), lambda i: (0, 0)),
            pl.BlockSpec((1, h), lambda i: (0, 0)),
            pl.BlockSpec((d, h), lambda i: (0, 0)),
            pl.BlockSpec((1, h), lambda i: (0, 0)),
            pl.BlockSpec((h, h), lambda i: (0, 0)),
            pl.BlockSpec((1, h), lambda i: (0, 0)),
            pl.BlockSpec((h, out), lambda i: (0, 0)),
            pl.BlockSpec((1, out), lambda i: (0, 0)),
        ],
        out_specs=pl.BlockSpec((blk, out), lambda i: (i, 0)),
        out_shape=jax.ShapeDtypeStruct((n, out), jnp.float32),
    )(x, aggp, cntp, Wl, bl.reshape(1, -1), Wr, br.reshape(1, -1),
      Wp1, bp1.reshape(1, -1), Wp2, bp2.reshape(1, -1))


def kernel(data, edge_index, W_l0, b_l0, W_r0, b_r0, W_l1, b_l1, W_r1, b_r1,
           W_p1, b_p1, W_p2, b_p2):
    src = edge_index[0]
    dst = edge_index[1]

    n = data.shape[0]
    agg0, cnt_raw = _segment_sum_sc(data, src, dst, with_cnt=True)
    cnt = cnt_raw.reshape(_NC * _NS, _HR * _HC)[:, :n].T
    x1 = _sage_dense(data, agg0, cnt, W_l0, b_l0, W_r0, b_r0, blk=2000)
    (agg1,) = _segment_sum_sc(x1, src, dst, with_cnt=False)
    return _sage_dense_post(x1, agg1, cnt, W_l1, b_l1, W_r1, b_r1,
                            W_p1, b_p1, W_p2, b_p2, blk=2000)
